# Initial kernel scaffold; baseline (speedup 1.0000x reference)
import jax
import jax.numpy as jnp
from jax.experimental import pallas as pl


def kernel(x, edge_index, eps, W1, b1, W2, b2, W3, b3, W4, b4):
    raise NotImplementedError("write your pallas kernel here")


# speedup vs baseline: 7.3434x; 7.3434x over previous
import functools

import jax
import jax.numpy as jnp
from jax import lax
from jax.experimental import pallas as pl
from jax.experimental.pallas import tpu as pltpu
from jax.experimental.pallas import tpu_sc as plsc

N_NODES = 10000
N_EDGES = 320000
D = 128

NC = 2
NS = 16
NW = NC * NS
EPW = N_EDGES // NW
CHUNK = 80
NCHUNK = EPW // CHUNK
N_PAD = 10240
RPS = N_PAD // NS

def _sc_agg_body(src_hbm, dst_hbm, x_hbm, zeros_hbm, out_hbm,
                 src_v, dst_v, rows_v, agg_sh, sem):
    c = lax.axis_index("c")
    s = lax.axis_index("s")
    wid = c * NS + s
    pltpu.sync_copy(src_hbm.at[wid], src_v)
    pltpu.sync_copy(dst_hbm.at[wid], dst_v)
    pltpu.sync_copy(zeros_hbm.at[pl.ds(s * RPS, RPS)],
                    agg_sh.at[pl.ds(s * RPS, RPS)])
    plsc.subcore_barrier()

    def body(j, carry):
        pltpu.async_copy(x_hbm.at[src_v.at[j]], rows_v, sem).wait()
        pltpu.sync_copy(rows_v, agg_sh.at[dst_v.at[j]], add=True)
        return carry

    lax.fori_loop(0, NCHUNK, body, 0)
    plsc.subcore_barrier()
    pltpu.sync_copy(agg_sh.at[pl.ds(s * RPS, RPS)],
                    out_hbm.at[c].at[pl.ds(s * RPS, RPS)])


@functools.cache
def _sc_agg():
    mesh = plsc.VectorSubcoreMesh(core_axis_name="c", subcore_axis_name="s",
                                  num_cores=NC, num_subcores=NS)
    return pl.kernel(
        _sc_agg_body,
        out_type=jax.ShapeDtypeStruct((NC, N_PAD, D), jnp.float32),
        mesh=mesh,
        scratch_types=[
            pltpu.VMEM((NCHUNK, CHUNK), jnp.int32),
            pltpu.VMEM((NCHUNK, CHUNK), jnp.int32),
            pltpu.VMEM((CHUNK, D), jnp.float32),
            pltpu.VMEM_SHARED((N_PAD, D), jnp.float32),
            pltpu.SemaphoreType.DMA,
        ],
    )


def _tc_mlp_body(eps_ref, x_ref, p0_ref, p1_ref,
                 W1_ref, b1_ref, W2_ref, b2_ref,
                 W3_ref, b3_ref, W4_ref, b4_ref, o_ref):
    h = (1.0 + eps_ref[0]) * x_ref[...] + p0_ref[...] + p1_ref[...]
    h = jnp.dot(h, W1_ref[...], preferred_element_type=jnp.float32)
    h = jnp.maximum(h + b1_ref[...], 0.0)
    h = jnp.dot(h, W2_ref[...], preferred_element_type=jnp.float32) + b2_ref[...]
    h = jnp.dot(h, W3_ref[...], preferred_element_type=jnp.float32)
    h = jnp.maximum(h + b3_ref[...], 0.0)
    h = jnp.dot(h, W4_ref[...], preferred_element_type=jnp.float32) + b4_ref[...]
    o_ref[...] = jax.nn.sigmoid(h)


BLK = 1000


def _tc_mlp(eps, x, p0, p1, W1, b1, W2, b2, W3, b3, W4, b4):
    wspec = pl.BlockSpec((D, D), lambda i: (0, 0))
    bspec = pl.BlockSpec((1, D), lambda i: (0, 0))
    rspec = pl.BlockSpec((BLK, D), lambda i: (i, 0))
    return pl.pallas_call(
        _tc_mlp_body,
        grid=(N_NODES // BLK,),
        in_specs=[
            pl.BlockSpec(memory_space=pltpu.SMEM),
            rspec, rspec, rspec,
            wspec, bspec, wspec, bspec,
            wspec, bspec, wspec, bspec,
        ],
        out_specs=rspec,
        out_shape=jax.ShapeDtypeStruct((N_NODES, D), jnp.float32),
    )(eps, x, p0, p1, W1, b1, W2, b2, W3, b3, W4, b4)


def kernel(x, edge_index, eps, W1, b1, W2, b2, W3, b3, W4, b4):
    src = edge_index[0].astype(jnp.int32).reshape(NW, NCHUNK, CHUNK)
    dst = edge_index[1].astype(jnp.int32).reshape(NW, NCHUNK, CHUNK)
    zeros = jnp.zeros((N_PAD, D), jnp.float32)
    parts = _sc_agg()(src, dst, x, zeros)
    eps1 = jnp.reshape(eps, (1,)).astype(jnp.float32)
    return _tc_mlp(eps1, x, parts[0, :N_NODES], parts[1, :N_NODES],
                   W1, b1.reshape(1, D), W2, b2.reshape(1, D),
                   W3, b3.reshape(1, D), W4, b4.reshape(1, D))
